# baseline (speedup 1.0000x reference)
import functools

import jax
import jax.numpy as jnp
from jax import lax
from jax.experimental import pallas as pl
from jax.experimental.pallas import tpu as pltpu
from jax.experimental.pallas import tpu_sc as plsc

_NC = 2
_NS = 16


_DW = 16


_D = 4


def _segment_sum_sc(table, src, dst, zeros, deg_aux=None):
    n_rows, width = table.shape
    dt = table.dtype
    n_chunks, chunk = src.shape
    n_acc = n_rows + _DW
    cpw = n_chunks // (_NC * _NS)
    rpt = n_rows // _NS
    ngrp = cpw // _D
    assert n_rows % _NS == 0 and n_chunks % (_NC * _NS) == 0
    assert cpw % _D == 0 and ngrp >= 3
    with_deg = deg_aux is not None

    mesh = plsc.VectorSubcoreMesh(core_axis_name="c", subcore_axis_name="s")

    out_type = [jax.ShapeDtypeStruct((_NC, n_rows, width), dt)]
    scratch = (
        [pltpu.VMEM((chunk,), jnp.int32) for _ in range(_D)]
        + [pltpu.VMEM((cpw, chunk), jnp.int32)]
        + [pltpu.VMEM((chunk, width), dt) for _ in range(_D)]
        + [pltpu.VMEM_SHARED((n_acc, width), dt)]
        + [pltpu.SemaphoreType.DMA] * (3 * _D)
    )
    if with_deg:
        out_type.append(jax.ShapeDtypeStruct((_NC, n_rows, _DW), jnp.float32))
        scratch += (
            [pltpu.VMEM((chunk, _DW), jnp.float32)]
            + [pltpu.VMEM_SHARED((n_acc, _DW), jnp.float32)]
            + [pltpu.SemaphoreType.DMA] * _D
        )

    @functools.partial(
        pl.kernel,
        out_type=out_type,
        mesh=mesh,
        scratch_types=scratch,
        compiler_params=pltpu.CompilerParams(use_tc_tiling_on_sc=False),
    )
    def seg_sum(*refs):
        if with_deg:
            (table_hbm, src_hbm, dst_hbm, zeros_hbm, zd_hbm, ones_hbm,
             out_hbm, outd_hbm) = refs[:8]
            rest = refs[8:]
        else:
            (table_hbm, src_hbm, dst_hbm, zeros_hbm, out_hbm) = refs[:5]
            rest = refs[5:]
        sidx = list(rest[:_D])
        didx = rest[_D]
        rows = list(rest[_D + 1:2 * _D + 1])
        accum = rest[2 * _D + 1]
        gsem = list(rest[2 * _D + 2:3 * _D + 2])
        ssem = list(rest[3 * _D + 2:4 * _D + 2])
        isem = list(rest[4 * _D + 2:5 * _D + 2])
        if with_deg:
            ones_rows = rest[5 * _D + 2]
            dacc = rest[5 * _D + 3]
            usem = list(rest[5 * _D + 4:6 * _D + 4])

        c = lax.axis_index("c")
        s = lax.axis_index("s")
        wid = c * _NS + s
        base = wid * cpw
        pltpu.sync_copy(zeros_hbm.at[pl.ds(s * rpt, rpt)],
                        accum.at[pl.ds(s * rpt, rpt)])
        if with_deg:
            pltpu.sync_copy(zd_hbm.at[pl.ds(s * rpt, rpt)],
                            dacc.at[pl.ds(s * rpt, rpt)])
            pltpu.sync_copy(ones_hbm, ones_rows)
        pltpu.sync_copy(dst_hbm.at[pl.ds(base, cpw)], didx)
        pltpu.sync_copy(src_hbm.at[base], sidx[0])
        pltpu.sync_copy(src_hbm.at[base + 1], sidx[1])
        plsc.subcore_barrier()

        def prefetch(n, b):
            pltpu.async_copy(src_hbm.at[base + n], sidx[b], isem[b])

        def wait_prefetch(n, b):
            pltpu.make_async_copy(src_hbm.at[base + n], sidx[b],
                                  isem[b]).wait()

        def gather(b):
            pltpu.async_copy(table_hbm.at[sidx[b]], rows[b], gsem[b])

        def wait_gather(b):
            pltpu.make_async_copy(table_hbm.at[sidx[b]], rows[b],
                                  gsem[b]).wait()

        def scatter(n, b):
            pltpu.async_copy(rows[b], accum.at[didx.at[n]], ssem[b],
                             add=True)
            if with_deg:
                pltpu.async_copy(ones_rows, dacc.at[didx.at[n]], usem[b],
                                 add=True)

        def wait_scatter(n, b):
            pltpu.make_async_copy(rows[b], accum.at[didx.at[n]],
                                  ssem[b]).wait()
            if with_deg:
                pltpu.make_async_copy(ones_rows, dacc.at[didx.at[n]],
                                      usem[b]).wait()

        def step(n, b, do_pref, do_gather, do_waits):
            wait_gather(b)
            if do_pref:
                prefetch(n + _D, b)
            scatter(n, b)
            if do_gather:
                b2 = (b + 2) % _D
                if do_waits:
                    wait_scatter(n - 2, b2)
                wait_prefetch(n + 2, b2)
                gather(b2)

        gather(0)
        gather(1)
        prefetch(2, 2)
        prefetch(3, 3)
        step(0, 0, True, True, False)
        step(1, 1, True, True, False)
        step(2, 2, True, True, True)
        step(3, 3, True, True, True)

        def body(m, carry):
            n0 = _D * m
            for b in range(_D):
                step(n0 + b, b, True, True, True)
            return carry

        lax.fori_loop(1, ngrp - 1, body, 0, unroll=False)
        n0 = cpw - _D
        step(n0, 0, False, True, True)
        step(n0 + 1, 1, False, True, True)
        step(n0 + 2, 2, False, False, False)
        step(n0 + 3, 3, False, False, False)
        for i in range(_D):
            wait_scatter(n0 + i, i)
        plsc.subcore_barrier()
        pltpu.sync_copy(accum.at[pl.ds(s * rpt, rpt)],
                        out_hbm.at[c, pl.ds(s * rpt, rpt)])
        if with_deg:
            pltpu.sync_copy(dacc.at[pl.ds(s * rpt, rpt)],
                            outd_hbm.at[c, pl.ds(s * rpt, rpt)])

    if with_deg:
        return seg_sum(table, src, dst, zeros, *deg_aux)
    return seg_sum(table, src, dst, zeros)


def _mid_tc(x, p0, pd, w_self0, w_neigh0, b0, w_neigh1, w_self1, b1):
    n, d_in = x.shape
    d_hid = w_self0.shape[1]
    d_out = w_neigh1.shape[1]
    blk = 1000
    grid = n // blk

    def body(x_ref, p_ref, pd_ref, ws_ref, wn_ref, b_ref, wn1_ref,
             ws1_ref, b1_ref, y0_ref, g_ref, invd_ref):
        acc = p_ref[0].astype(jnp.float32) + p_ref[1].astype(jnp.float32)
        deg = pd_ref[0, :, 0:1] + pd_ref[1, :, 0:1]
        inv = 1.0 / jnp.maximum(deg, 1.0)
        hn = acc * inv
        h1 = x_ref[...] @ ws_ref[...] + hn @ wn_ref[...] + b_ref[...]
        h1 = jnp.maximum(h1, 0.0)
        y0_ref[...] = h1 @ ws1_ref[...] + b1_ref[...]
        g_ref[...] = (h1 @ wn1_ref[...]).astype(g_ref.dtype)
        invd_ref[...] = inv

    return pl.pallas_call(
        body,
        grid=(grid,),
        in_specs=[
            pl.BlockSpec((blk, d_in), lambda i: (i, 0)),
            pl.BlockSpec((_NC, blk, d_in), lambda i: (0, i, 0)),
            pl.BlockSpec((_NC, blk, _DW), lambda i: (0, i, 0)),
            pl.BlockSpec((d_in, d_hid), lambda i: (0, 0)),
            pl.BlockSpec((d_in, d_hid), lambda i: (0, 0)),
            pl.BlockSpec((1, d_hid), lambda i: (0, 0)),
            pl.BlockSpec((d_hid, d_out), lambda i: (0, 0)),
            pl.BlockSpec((d_hid, d_out), lambda i: (0, 0)),
            pl.BlockSpec((1, d_out), lambda i: (0, 0)),
        ],
        out_specs=[
            pl.BlockSpec((blk, d_out), lambda i: (i, 0)),
            pl.BlockSpec((blk, d_out), lambda i: (i, 0)),
            pl.BlockSpec((blk, 1), lambda i: (i, 0)),
        ],
        out_shape=[
            jax.ShapeDtypeStruct((n, d_out), jnp.float32),
            jax.ShapeDtypeStruct((n, d_out), jnp.bfloat16),
            jax.ShapeDtypeStruct((n, 1), jnp.float32),
        ],
    )(x, p0, pd, w_self0, w_neigh0, b0, w_neigh1, w_self1, b1)


def _final_tc(y0, p1, inv_deg):
    n, d_out = y0.shape
    blk = 1000
    grid = n // blk

    def body(y_ref, p_ref, invd_ref, out_ref):
        agg = p_ref[0].astype(jnp.float32) + p_ref[1].astype(jnp.float32)
        out_ref[...] = y_ref[...] + agg * invd_ref[...]

    return pl.pallas_call(
        body,
        grid=(grid,),
        in_specs=[
            pl.BlockSpec((blk, d_out), lambda i: (i, 0)),
            pl.BlockSpec((_NC, blk, d_out), lambda i: (0, i, 0)),
            pl.BlockSpec((blk, 1), lambda i: (i, 0)),
        ],
        out_specs=pl.BlockSpec((blk, d_out), lambda i: (i, 0)),
        out_shape=jax.ShapeDtypeStruct((n, d_out), jnp.float32),
    )(y0, p1, inv_deg)


def kernel(x, edge_index, W_self0, W_neigh0, b0, W_self1, W_neigh1, b1):
    n, d_in = x.shape
    e = edge_index.shape[1]
    chunk = 128
    e_pad = -e % (_NC * _NS * _D * chunk)
    src = jnp.concatenate(
        [edge_index[0], jnp.zeros((e_pad,), jnp.int32)]).reshape(-1, chunk)
    dst = jnp.concatenate(
        [edge_index[1], jnp.full((e_pad,), n, jnp.int32)]).reshape(-1, chunk)

    zeros = jnp.zeros((n, d_in), jnp.bfloat16)
    zeros_d = jnp.zeros((n, _DW), jnp.float32)
    ones_blk = jnp.ones((chunk, _DW), jnp.float32)

    p0, pd = _segment_sum_sc(x.astype(jnp.bfloat16), src, dst, zeros,
                             deg_aux=(zeros_d, ones_blk))
    y0, g, inv_deg = _mid_tc(x, p0, pd, W_self0, W_neigh0,
                             b0.reshape(1, -1), W_neigh1,
                             W_self1, b1.reshape(1, -1))
    p1, = _segment_sum_sc(g, src, dst, zeros)
    return _final_tc(y0, p1, inv_deg)
